# baseline (speedup 1.0000x reference)
import functools

import jax
import jax.numpy as jnp
from jax import lax
from jax.experimental import pallas as pl
from jax.experimental.pallas import tpu as pltpu
from jax.experimental.pallas import tpu_sc as plsc

_THRESHOLD = 0.01


def _make_sc(M, rows, hw):
    NW = 32
    rpw = rows // NW
    nct = hw // 16
    mesh = plsc.VectorSubcoreMesh(core_axis_name="c", subcore_axis_name="s")

    @functools.partial(
        pl.kernel,
        mesh=mesh,
        compiler_params=pltpu.CompilerParams(needs_layout_passes=False),
        out_type=jax.ShapeDtypeStruct((M, rows, hw), jnp.float32),
        scratch_types=[
            pltpu.VMEM((hw,), jnp.float32),
            pltpu.VMEM((hw,), jnp.float32),
        ],
    )
    def k(x_hbm, o_hbm, b0, b1):
        wid = lax.axis_index("s") * 2 + lax.axis_index("c")
        row0 = wid * rpw

        def row_body(i, carry):
            row = row0 + i
            pltpu.sync_copy(x_hbm.at[0, row], b0)
            pltpu.sync_copy(x_hbm.at[1, row], b1)

            def acc_body(ct, c):
                a0, q0, a1, q1 = c
                v0 = b0[pl.ds(ct * 16, 16)]
                v1 = b1[pl.ds(ct * 16, 16)]
                return (a0 + v0, q0 + v0 * v0, a1 + v1, q1 + v1 * v1)

            z = jnp.zeros((16,), jnp.float32)
            a0, q0, a1, q1 = lax.fori_loop(0, nct, acc_body, (z, z, z, z))
            s0 = jnp.sum(a0, axis=0)
            ss0 = jnp.sum(q0, axis=0)
            s1 = jnp.sum(a1, axis=0)
            ss1 = jnp.sum(q1, axis=0)
            var0 = (ss0 - s0 * s0 * (1.0 / hw)) * (1.0 / (hw - 1))
            var1 = (ss1 - s1 * s1 * (1.0 / hw)) * (1.0 / (hw - 1))
            m0 = var0 >= _THRESHOLD
            m1 = var1 >= _THRESHOLD

            @pl.when(m0)
            def _():
                pltpu.sync_copy(b0, o_hbm.at[0, row])

            @pl.when(jnp.logical_not(m0))
            def _():
                pltpu.sync_copy(b1, o_hbm.at[0, row])

            @pl.when(m1)
            def _():
                pltpu.sync_copy(b1, o_hbm.at[1, row])

            @pl.when(jnp.logical_not(m1))
            def _():
                pltpu.sync_copy(b0, o_hbm.at[1, row])

            return carry

        lax.fori_loop(0, rpw, row_body, 0)

    return k


def kernel(x):
    M, n, c, H, W = x.shape
    rows, hw = n * c, H * W
    xs = x.reshape(M, rows, hw)
    out = _make_sc(M, rows, hw)(xs)
    return out.reshape(M, n, c, H, W)


# speedup vs baseline: 1.0947x; 1.0947x over previous
import functools

import jax
import jax.numpy as jnp
from jax import lax
from jax.experimental import pallas as pl
from jax.experimental.pallas import tpu as pltpu
from jax.experimental.pallas import tpu_sc as plsc

_THRESHOLD = 0.01


def _make_sc(M, rows, hw):
    NW = 32
    rpw = rows // NW
    nct = hw // 16
    RB = 8
    nb = rpw // RB
    mesh = plsc.VectorSubcoreMesh(core_axis_name="c", subcore_axis_name="s")

    @functools.partial(
        pl.kernel,
        mesh=mesh,
        compiler_params=pltpu.CompilerParams(needs_layout_passes=False),
        out_type=jax.ShapeDtypeStruct((M, rows, hw), jnp.float32),
        scratch_types=[
            pltpu.VMEM((RB, hw), jnp.float32),
            pltpu.VMEM((RB, hw), jnp.float32),
            pltpu.VMEM((RB, hw), jnp.float32),
            pltpu.VMEM((RB, hw), jnp.float32),
        ],
    )
    def k(x_hbm, o_hbm, b0, b1, c0, c1):
        wid = lax.axis_index("s") * 2 + lax.axis_index("c")
        row0 = wid * rpw

        def batch_body(ib, carry):
            r0 = row0 + ib * RB
            pltpu.sync_copy(x_hbm.at[0, pl.ds(r0, RB)], b0)
            pltpu.sync_copy(x_hbm.at[1, pl.ds(r0, RB)], b1)
            for r in range(RB):
                def acc_body(ct, c):
                    a0, q0, a1, q1 = c
                    v0 = b0[r, pl.ds(ct * 16, 16)]
                    v1 = b1[r, pl.ds(ct * 16, 16)]
                    return (a0 + v0, q0 + v0 * v0, a1 + v1, q1 + v1 * v1)

                z = jnp.zeros((16,), jnp.float32)
                a0, q0, a1, q1 = lax.fori_loop(0, nct, acc_body, (z, z, z, z))
                s0 = jnp.sum(a0, axis=0)
                ss0 = jnp.sum(q0, axis=0)
                s1 = jnp.sum(a1, axis=0)
                ss1 = jnp.sum(q1, axis=0)
                var0 = (ss0 - s0 * s0 * (1.0 / hw)) * (1.0 / (hw - 1))
                var1 = (ss1 - s1 * s1 * (1.0 / hw)) * (1.0 / (hw - 1))
                m0 = jnp.broadcast_to(var0 >= _THRESHOLD, (16,))
                m1 = jnp.broadcast_to(var1 >= _THRESHOLD, (16,))

                def sel_body(ct, c):
                    v0 = b0[r, pl.ds(ct * 16, 16)]
                    v1 = b1[r, pl.ds(ct * 16, 16)]
                    c0[r, pl.ds(ct * 16, 16)] = jnp.where(m0, v0, v1)
                    c1[r, pl.ds(ct * 16, 16)] = jnp.where(m1, v1, v0)
                    return c

                lax.fori_loop(0, nct, sel_body, 0)
            pltpu.sync_copy(c0, o_hbm.at[0, pl.ds(r0, RB)])
            pltpu.sync_copy(c1, o_hbm.at[1, pl.ds(r0, RB)])
            return carry

        lax.fori_loop(0, nb, batch_body, 0)

    return k


def kernel(x):
    M, n, c, H, W = x.shape
    rows, hw = n * c, H * W
    xs = x.reshape(M, rows, hw)
    out = _make_sc(M, rows, hw)(xs)
    return out.reshape(M, n, c, H, W)


# speedup vs baseline: 1.1120x; 1.0158x over previous
import functools

import jax
import jax.numpy as jnp
from jax import lax
from jax.experimental import pallas as pl
from jax.experimental.pallas import tpu as pltpu
from jax.experimental.pallas import tpu_sc as plsc

_THRESHOLD = 0.01


def _make_sc(M, rows, hw):
    NW = 32
    rpw = rows // NW
    nct = hw // 16
    RB = 8
    nb = rpw // RB
    mesh = plsc.VectorSubcoreMesh(core_axis_name="c", subcore_axis_name="s")

    @functools.partial(
        pl.kernel,
        mesh=mesh,
        compiler_params=pltpu.CompilerParams(needs_layout_passes=False),
        out_type=jax.ShapeDtypeStruct((M, rows, hw), jnp.float32),
        scratch_types=[
            pltpu.VMEM((RB, hw), jnp.float32),
            pltpu.VMEM((RB, hw), jnp.float32),
            pltpu.VMEM((RB, hw), jnp.float32),
            pltpu.VMEM((RB, hw), jnp.float32),
        ],
    )
    def k(x_hbm, o_hbm, b0, b1, c0, c1):
        wid = lax.axis_index("s") * 2 + lax.axis_index("c")
        row0 = wid * rpw

        def batch_body(ib, carry):
            r0 = row0 + ib * RB
            pltpu.sync_copy(x_hbm.at[0, pl.ds(r0, RB)], b0)
            pltpu.sync_copy(x_hbm.at[1, pl.ds(r0, RB)], b1)
            for r in range(RB):
                def acc_body(ct, c):
                    a0, q0, a1, q1 = c
                    v0 = b0[r, pl.ds(ct * 16, 16)]
                    v1 = b1[r, pl.ds(ct * 16, 16)]
                    return (a0 + v0, q0 + v0 * v0, a1 + v1, q1 + v1 * v1)

                z = jnp.zeros((16,), jnp.float32)
                a0, q0, a1, q1 = lax.fori_loop(0, nct, acc_body, (z, z, z, z), unroll=7)
                s0 = jnp.sum(a0, axis=0)
                ss0 = jnp.sum(q0, axis=0)
                s1 = jnp.sum(a1, axis=0)
                ss1 = jnp.sum(q1, axis=0)
                var0 = (ss0 - s0 * s0 * (1.0 / hw)) * (1.0 / (hw - 1))
                var1 = (ss1 - s1 * s1 * (1.0 / hw)) * (1.0 / (hw - 1))
                m0 = jnp.broadcast_to(var0 >= _THRESHOLD, (16,))
                m1 = jnp.broadcast_to(var1 >= _THRESHOLD, (16,))

                def sel_body(ct, c):
                    v0 = b0[r, pl.ds(ct * 16, 16)]
                    v1 = b1[r, pl.ds(ct * 16, 16)]
                    c0[r, pl.ds(ct * 16, 16)] = jnp.where(m0, v0, v1)
                    c1[r, pl.ds(ct * 16, 16)] = jnp.where(m1, v1, v0)
                    return c

                lax.fori_loop(0, nct, sel_body, 0, unroll=7)
            pltpu.sync_copy(c0, o_hbm.at[0, pl.ds(r0, RB)])
            pltpu.sync_copy(c1, o_hbm.at[1, pl.ds(r0, RB)])
            return carry

        lax.fori_loop(0, nb, batch_body, 0)

    return k


def kernel(x):
    M, n, c, H, W = x.shape
    rows, hw = n * c, H * W
    xs = x.reshape(M, rows, hw)
    out = _make_sc(M, rows, hw)(xs)
    return out.reshape(M, n, c, H, W)


# speedup vs baseline: 14.2416x; 12.8075x over previous
import jax
import jax.numpy as jnp
from jax.experimental import pallas as pl

_THRESHOLD = 0.01


def _tc_body(x_ref, o_ref):
    x0 = x_ref[0, 0]
    x1 = x_ref[1, 0]
    hw = x0.shape[0] * x0.shape[1]
    s0 = jnp.sum(x0, axis=(0, 1), keepdims=True)
    s1 = jnp.sum(x1, axis=(0, 1), keepdims=True)
    ss0 = jnp.sum(x0 * x0, axis=(0, 1), keepdims=True)
    ss1 = jnp.sum(x1 * x1, axis=(0, 1), keepdims=True)
    v0 = (ss0 - s0 * s0 * (1.0 / hw)) * (1.0 / (hw - 1))
    v1 = (ss1 - s1 * s1 * (1.0 / hw)) * (1.0 / (hw - 1))
    o_ref[0, 0] = jnp.where(v0 >= _THRESHOLD, x0, x1)
    o_ref[1, 0] = jnp.where(v1 >= _THRESHOLD, x1, x0)


def kernel(x):
    M, n, c, H, W = x.shape
    xt = jnp.transpose(x, (0, 1, 3, 4, 2))
    CB = 384
    out_t = pl.pallas_call(
        _tc_body,
        grid=(n, c // CB),
        in_specs=[pl.BlockSpec((M, 1, H, W, CB), lambda i, j: (0, i, 0, 0, j))],
        out_specs=pl.BlockSpec((M, 1, H, W, CB), lambda i, j: (0, i, 0, 0, j)),
        out_shape=jax.ShapeDtypeStruct((M, n, H, W, c), jnp.float32),
    )(xt)
    return jnp.transpose(out_t, (0, 1, 4, 2, 3))
